# baseline (speedup 1.0000x reference)
import functools

import jax
import jax.numpy as jnp
from jax import lax
from jax.experimental import pallas as pl
from jax.experimental.pallas import tpu as pltpu
from jax.experimental.pallas import tpu_sc as plsc

NC = 2
NS = 16
NW = NC * NS

OWN_SHIFT = 15
SPAN = 1 << OWN_SHIFT
PW = 512
TCAP = 64
CCAP = 192
PTCAP = 32
PCCAP = 112


def _lane0(lanes):
    return lanes == 0


def _store1(ref, pos, val, lanes):
    plsc.store_scatter(ref, [jnp.full((16,), pos, jnp.int32)],
                       jnp.full((16,), val, ref.dtype), mask=_lane0(lanes))


def _load1(ref, *pos):
    idx = [jnp.full((16,), p, jnp.int32) for p in pos]
    return plsc.load_gather(ref, idx)[0]


def _store1_2d(ref, col, val, lanes):
    plsc.store_scatter(ref,
                       [jnp.zeros((16,), jnp.int32),
                        jnp.full((16,), col, jnp.int32)],
                       jnp.full((16,), val, ref.dtype), mask=_lane0(lanes))


def _bucketize_body(tgt_hbm, ctx_hbm, tb_hbm, cb_hbm, out_hbm,
                    tv, cv, st_t, st_c, cnt_t, cnt_c, sem,
                    *, b_per_w, c_dim):
    wid = lax.axis_index("s") * NC + lax.axis_index("c")
    base = wid * b_per_w
    lanes = lax.iota(jnp.int32, 16)
    pltpu.sync_copy(tgt_hbm.at[pl.ds(base, b_per_w)], tv)
    pltpu.sync_copy(ctx_hbm.at[:, pl.ds(base, b_per_w)], cv)

    def zero_cnt(i, carry):
        cnt_t[i] = 0
        cnt_c[i] = 0
        return carry
    lax.fori_loop(0, 32, zero_cnt, 0)

    def req(i, carry):
        v = _load1(tv, i)
        o = jnp.minimum(lax.shift_right_logical(v, OWN_SHIFT), 30)
        k = cnt_t[o]
        _store1(st_t, o * (2 * TCAP) + 2 * k, v, lanes)
        _store1(st_t, o * (2 * TCAP) + 2 * k + 1, base + i, lanes)
        cnt_t[o] = k + 1
        for c in range(c_dim):
            v2 = _load1(cv, c, i)
            o2 = jnp.minimum(lax.shift_right_logical(v2, OWN_SHIFT), 30)
            k2 = cnt_c[o2]
            _store1(st_c, o2 * (2 * CCAP) + 2 * k2, v2, lanes)
            _store1(st_c, o2 * (2 * CCAP) + 2 * k2 + 1,
                    c * (b_per_w * NW) + base + i, lanes)
            cnt_c[o2] = k2 + 1
        return carry
    lax.fori_loop(0, b_per_w, req, 0)

    def pub(o, carry):
        _store1(st_t, o * (2 * TCAP) + 2 * TCAP - 2, cnt_t[o], lanes)
        _store1(st_c, o * (2 * CCAP) + 2 * CCAP - 2, cnt_c[o], lanes)
        return carry
    lax.fori_loop(0, 32, pub, 0)

    pltpu.sync_copy(st_t, tb_hbm.at[wid])
    pltpu.sync_copy(st_c, cb_hbm.at[wid])
    out_v = tv
    pltpu.sync_copy(out_v.at[pl.ds(0, 8)], out_hbm.at[pl.ds(wid * 8, 8)])


def _sweep_body(ttab, ctab, ttail, ctail, tb_hbm, cb_hbm, te_g, ce_g,
                panel, tailp, rq_t, rq_c, pt, pc, trow, trow2, crow,
                crow2, tdst, tdst2, cdst, cdst2, pcnt_t, pcnt_c,
                sem, psemA, ssemA, ssemB, *, e_dim, b_dim, c_dim):
    ssems = (ssemA, ssemB)
    o = lax.axis_index("s") * NC + lax.axis_index("c")
    lanes = lax.iota(jnp.int32, 16)
    vbase = o * SPAN
    dump_t = jnp.int32(b_dim)
    dump_c = jnp.int32(b_dim * c_dim)

    pltpu.sync_copy(tb_hbm.at[:, pl.ds(o * (2 * TCAP), 2 * TCAP)], rq_t)
    pltpu.sync_copy(cb_hbm.at[:, pl.ds(o * (2 * CCAP), 2 * CCAP)], rq_c)

    def zc(i, carry):
        pcnt_t[i] = 0
        pcnt_c[i] = 0
        return carry
    lax.fori_loop(0, 64, zc, 0)

    def reb(w, carry):
        nt = _load1(rq_t, w, 2 * TCAP - 2)
        nc_ = _load1(rq_c, w, 2 * CCAP - 2)

        def bt(i, c2):
            v = _load1(rq_t, w, 2 * i)
            dst = _load1(rq_t, w, 2 * i + 1)
            dv = v - vbase
            p = lax.shift_right_logical(dv, 9)
            k = pcnt_t[p]
            _store1(pt, p * (2 * PTCAP) + 2 * k, dv - p * PW, lanes)
            _store1(pt, p * (2 * PTCAP) + 2 * k + 1, dst, lanes)
            pcnt_t[p] = k + 1
            return c2
        lax.fori_loop(0, nt, bt, 0)

        def bc(i, c2):
            v = _load1(rq_c, w, 2 * i)
            dst = _load1(rq_c, w, 2 * i + 1)
            dv = v - vbase
            p = lax.shift_right_logical(dv, 9)
            k = pcnt_c[p]
            _store1(pc, p * (2 * PCCAP) + 2 * k, dv - p * PW, lanes)
            _store1(pc, p * (2 * PCCAP) + 2 * k + 1, dst, lanes)
            pcnt_c[p] = k + 1
            return c2
        lax.fori_loop(0, nc_, bc, 0)
        return carry
    lax.fori_loop(0, NW, reb, 0)

    npan_full = jnp.where(o < 30, 64, jnp.where(o == 30, 33, 0))
    ng = e_dim // 16

    def fill_panel(tab, p, pan):
        cps = []
        for r in range(e_dim // 8):
            cps.append(pltpu.async_copy(
                tab.at[pl.ds(r * 8, 8), pl.ds(vbase + p * PW, PW)],
                pan.at[pl.ds(r * 8, 8), :], psemA))
        for cp in cps:
            cp.wait()

    def serve(g_hbm, pbuf, cap, row, dbuf, dump, cnt_ref, pan, ssem,
              defer):
        def inner(p, carry):
            if defer:
                @pl.when(p >= 2)
                def _():
                    pltpu.make_async_copy(row, g_hbm.at[dbuf.at[0]],
                                          ssem).wait()
            nslot = cap // 16
            for s in range(nslot):
                dbuf[0, pl.ds(s * 16, 16)] = jnp.full((16,), dump, jnp.int32)
            n = cnt_ref[p]

            def one(i, c2):
                dv = _load1(pbuf, p * (2 * cap) + 2 * i)
                dst = _load1(pbuf, p * (2 * cap) + 2 * i + 1)
                for g in range(ng):
                    col = plsc.load_gather(
                        pan, [g * 16 + lanes, jnp.full((16,), dv,
                                                       jnp.int32)])
                    row[i, pl.ds(g * 16, 16)] = col
                _store1_2d(dbuf, i, dst, lanes)
                return c2
            lax.fori_loop(0, n, one, 0)
            cp = pltpu.async_copy(row, g_hbm.at[dbuf.at[0]], ssem)
            if not defer:
                cp.wait()
            return carry
        return inner

    def sweep_table(tab, g_hbm, pbuf, cap, rows, dbufs, dump, cnt_ref):
        srv = [serve(g_hbm, pbuf, cap, rows[b], dbufs[b], dump, cnt_ref,
                     panel, ssems[b], True) for b in (0, 1)]

        def two(p2, carry):
            p = p2 * 2

            @pl.when(p < npan_full)
            def _():
                fill_panel(tab, p, panel)
                srv[0](p, 0)

            @pl.when(p + 1 < npan_full)
            def _():
                fill_panel(tab, p + 1, panel)
                srv[1](p + 1, 0)
            return carry
        lax.fori_loop(0, 32, two, 0)
        for b in (0, 1):
            @pl.when(npan_full >= b + 1)
            def _():
                pltpu.make_async_copy(rows[b], g_hbm.at[dbufs[b].at[0]],
                                      ssems[b]).wait()

    sweep_table(ttab, te_g, pt, PTCAP, (trow, trow2), (tdst, tdst2),
                dump_t, pcnt_t)
    sweep_table(ctab, ce_g, pc, PCCAP, (crow, crow2), (cdst, cdst2),
                dump_c, pcnt_c)

    @pl.when(o == 30)
    def _tail():
        pltpu.sync_copy(ttail, tailp)
        serve(te_g, pt, PTCAP, trow, tdst, dump_t, pcnt_t, tailp,
              ssems[0], False)(33, 0)
        pltpu.sync_copy(ctail, tailp)
        serve(ce_g, pc, PCCAP, crow, cdst, dump_c, pcnt_c, tailp,
              ssems[1], False)(33, 0)


def _dots_body(te_hbm, ce_hbm, out_hbm, te_v, ce_v, out_v, sem,
               *, b_per_w, b_dim, c_dim, e_dim):
    wid = lax.axis_index("s") * NC + lax.axis_index("c")
    lanes = lax.iota(jnp.int32, 16)
    cb = 128
    for chunk in range(b_per_w // cb):
        base = wid * b_per_w + chunk * cb
        copies = [pltpu.async_copy(te_hbm.at[pl.ds(base, cb)], te_v, sem)]
        for c in range(c_dim):
            copies.append(pltpu.async_copy(
                ce_hbm.at[pl.ds(c * b_dim + base, cb)],
                ce_v.at[pl.ds(c * cb, cb)], sem))
        for cp in copies:
            cp.wait()

        for blk in range(cb // 16):
            rows16 = blk * 16 + lanes
            zero = jnp.zeros((16,), jnp.float32)

            def ebody(e, accs):
                ecol = jnp.full((16,), e, jnp.int32)
                tg = plsc.load_gather(te_v, [rows16, ecol])
                return tuple(
                    accs[c] + tg * plsc.load_gather(
                        ce_v, [c * cb + rows16, ecol])
                    for c in range(c_dim))

            accs = lax.fori_loop(0, e_dim, ebody, (zero,) * c_dim)
            for c in range(c_dim):
                out_v[c, pl.ds(blk * 16, 16)] = accs[c]

        pltpu.sync_copy(out_v, out_hbm.at[:, pl.ds(base, cb)])


def kernel(target, context, target_table, context_table):
    b_dim = target.shape[0]
    c_dim = context.shape[1]
    e_dim = target_table.shape[1]
    b_per_w = b_dim // NW

    ctx_t = context.T
    tt_t = target_table.T
    ct_t = context_table.T

    mesh = plsc.VectorSubcoreMesh(core_axis_name="c", subcore_axis_name="s")
    params = pltpu.CompilerParams(needs_layout_passes=False)

    bucketize = functools.partial(
        pl.kernel, mesh=mesh, compiler_params=params,
        out_type=(
            jax.ShapeDtypeStruct((NW, 32 * 2 * TCAP), jnp.int32),
            jax.ShapeDtypeStruct((NW, 32 * 2 * CCAP), jnp.int32),
            jax.ShapeDtypeStruct((NW * 8,), jnp.int32),
        ),
        scratch_types=[
            pltpu.VMEM((b_per_w,), jnp.int32),
            pltpu.VMEM((c_dim, b_per_w), jnp.int32),
            pltpu.VMEM((32 * 2 * TCAP,), jnp.int32),
            pltpu.VMEM((32 * 2 * CCAP,), jnp.int32),
            pltpu.SMEM((32,), jnp.int32),
            pltpu.SMEM((32,), jnp.int32),
            pltpu.SemaphoreType.DMA,
        ],
    )(functools.partial(_bucketize_body, b_per_w=b_per_w, c_dim=c_dim))
    tb, cb_, _sent = bucketize(target, ctx_t)

    sweep = functools.partial(
        pl.kernel, mesh=mesh, compiler_params=params,
        out_type=(
            jax.ShapeDtypeStruct((b_dim + 1, 128), jnp.float32),
            jax.ShapeDtypeStruct((b_dim * c_dim + 1, 128), jnp.float32),
        ),
        scratch_types=[
            pltpu.VMEM((e_dim, PW), jnp.float32),
            pltpu.VMEM((e_dim, 64), jnp.float32),
            pltpu.VMEM((32, 2 * TCAP), jnp.int32),
            pltpu.VMEM((32, 2 * CCAP), jnp.int32),
            pltpu.VMEM((64 * 2 * PTCAP,), jnp.int32),
            pltpu.VMEM((64 * 2 * PCCAP,), jnp.int32),
            pltpu.VMEM((PTCAP, 128), jnp.float32),
            pltpu.VMEM((PTCAP, 128), jnp.float32),
            pltpu.VMEM((PCCAP, 128), jnp.float32),
            pltpu.VMEM((PCCAP, 128), jnp.float32),
            pltpu.VMEM((1, PTCAP), jnp.int32),
            pltpu.VMEM((1, PTCAP), jnp.int32),
            pltpu.VMEM((1, PCCAP), jnp.int32),
            pltpu.VMEM((1, PCCAP), jnp.int32),
            pltpu.SMEM((64,), jnp.int32),
            pltpu.SMEM((64,), jnp.int32),
            pltpu.SemaphoreType.DMA,
            pltpu.SemaphoreType.DMA,
            pltpu.SemaphoreType.DMA,
            pltpu.SemaphoreType.DMA,
        ],
    )(functools.partial(_sweep_body, e_dim=e_dim, b_dim=b_dim, c_dim=c_dim))
    vfull = (SPAN * 30) + ((target_table.shape[0] - SPAN * 30) // PW) * PW
    tt_tail = target_table[vfull:, :].T
    ct_tail = context_table[vfull:, :].T
    te_g, ce_g = sweep(tt_t, ct_t, tt_tail, ct_tail, tb, cb_)

    dots = functools.partial(
        pl.kernel, mesh=mesh, compiler_params=params,
        out_type=jax.ShapeDtypeStruct((c_dim, b_dim), jnp.float32),
        scratch_types=[
            pltpu.VMEM((128, 128), jnp.float32),
            pltpu.VMEM((c_dim * 128, 128), jnp.float32),
            pltpu.VMEM((c_dim, 128), jnp.float32),
            pltpu.SemaphoreType.DMA,
        ],
    )(functools.partial(_dots_body, b_per_w=b_per_w, b_dim=b_dim,
                        c_dim=c_dim, e_dim=e_dim))
    out = dots(te_g, ce_g)
    return out.T


# speedup vs baseline: 7.0871x; 7.0871x over previous
import functools

import jax
import jax.numpy as jnp
from jax import lax
from jax.experimental import pallas as pl
from jax.experimental.pallas import tpu as pltpu
from jax.experimental.pallas import tpu_sc as plsc

NC = 2
NS = 16
NW = NC * NS

OWN_SHIFT = 15
SPAN = 1 << OWN_SHIFT
PW = 512
TCAP = 64
CCAP = 192
PTCAP = 32
PCCAP = 112


def _lane0(lanes):
    return lanes == 0


def _store1(ref, pos, val, lanes):
    plsc.store_scatter(ref, [jnp.full((16,), pos, jnp.int32)],
                       jnp.full((16,), val, ref.dtype), mask=_lane0(lanes))


def _load1(ref, *pos):
    idx = [jnp.full((16,), p, jnp.int32) for p in pos]
    return plsc.load_gather(ref, idx)[0]


def _store1_2d(ref, col, val, lanes):
    plsc.store_scatter(ref,
                       [jnp.zeros((16,), jnp.int32),
                        jnp.full((16,), col, jnp.int32)],
                       jnp.full((16,), val, ref.dtype), mask=_lane0(lanes))


def _bucketize_body(tgt_hbm, ctx_hbm, tb_hbm, cb_hbm, out_hbm,
                    tv, cv, st_t, st_c, cnt_t, cnt_c, sem,
                    *, b_per_w, c_dim):
    wid = lax.axis_index("s") * NC + lax.axis_index("c")
    base = wid * b_per_w
    lanes = lax.iota(jnp.int32, 16)
    pltpu.sync_copy(tgt_hbm.at[pl.ds(base, b_per_w)], tv)
    pltpu.sync_copy(ctx_hbm.at[:, pl.ds(base, b_per_w)], cv)

    def zero_cnt(i, carry):
        cnt_t[i] = 0
        cnt_c[i] = 0
        return carry
    lax.fori_loop(0, 32, zero_cnt, 0)

    def req(i, carry):
        v = _load1(tv, i)
        o = jnp.minimum(lax.shift_right_logical(v, OWN_SHIFT), 30)
        k = cnt_t[o]
        _store1(st_t, o * (2 * TCAP) + 2 * k, v, lanes)
        _store1(st_t, o * (2 * TCAP) + 2 * k + 1, base + i, lanes)
        cnt_t[o] = k + 1
        for c in range(c_dim):
            v2 = _load1(cv, c, i)
            o2 = jnp.minimum(lax.shift_right_logical(v2, OWN_SHIFT), 30)
            k2 = cnt_c[o2]
            _store1(st_c, o2 * (2 * CCAP) + 2 * k2, v2, lanes)
            _store1(st_c, o2 * (2 * CCAP) + 2 * k2 + 1,
                    c * (b_per_w * NW) + base + i, lanes)
            cnt_c[o2] = k2 + 1
        return carry
    lax.fori_loop(0, b_per_w, req, 0)

    def pub(o, carry):
        _store1(st_t, o * (2 * TCAP) + 2 * TCAP - 2, cnt_t[o], lanes)
        _store1(st_c, o * (2 * CCAP) + 2 * CCAP - 2, cnt_c[o], lanes)
        return carry
    lax.fori_loop(0, 32, pub, 0)

    pltpu.sync_copy(st_t, tb_hbm.at[wid])
    pltpu.sync_copy(st_c, cb_hbm.at[wid])
    out_v = tv
    pltpu.sync_copy(out_v.at[pl.ds(0, 8)], out_hbm.at[pl.ds(wid * 8, 8)])


def _sweep_body(ttab, ctab, ttail, ctail, tb_hbm, cb_hbm, te_g, ce_g,
                panel, tailp, rq_t, rq_c, pt, pc, trow, trow2, crow,
                crow2, tdst, tdst2, cdst, cdst2, pcnt_t, pcnt_c,
                sem, psemA, ssemA, ssemB, *, e_dim, b_dim, c_dim):
    ssems = (ssemA, ssemB)
    o = lax.axis_index("s") * NC + lax.axis_index("c")
    lanes = lax.iota(jnp.int32, 16)
    vbase = o * SPAN
    dump_t = jnp.int32(b_dim)
    dump_c = jnp.int32(b_dim * c_dim)

    pltpu.sync_copy(tb_hbm.at[:, pl.ds(o * (2 * TCAP), 2 * TCAP)], rq_t)
    pltpu.sync_copy(cb_hbm.at[:, pl.ds(o * (2 * CCAP), 2 * CCAP)], rq_c)

    def zc(i, carry):
        pcnt_t[i] = 0
        pcnt_c[i] = 0
        return carry
    lax.fori_loop(0, 64, zc, 0)

    def reb(w, carry):
        nt = _load1(rq_t, w, 2 * TCAP - 2)
        nc_ = _load1(rq_c, w, 2 * CCAP - 2)

        def bt(i, c2):
            v = _load1(rq_t, w, 2 * i)
            dst = _load1(rq_t, w, 2 * i + 1)
            dv = v - vbase
            p = lax.shift_right_logical(dv, 9)
            k = pcnt_t[p]
            _store1(pt, p * (2 * PTCAP) + 2 * k, dv - p * PW, lanes)
            _store1(pt, p * (2 * PTCAP) + 2 * k + 1, dst, lanes)
            pcnt_t[p] = k + 1
            return c2
        lax.fori_loop(0, nt, bt, 0)

        def bc(i, c2):
            v = _load1(rq_c, w, 2 * i)
            dst = _load1(rq_c, w, 2 * i + 1)
            dv = v - vbase
            p = lax.shift_right_logical(dv, 9)
            k = pcnt_c[p]
            _store1(pc, p * (2 * PCCAP) + 2 * k, dv - p * PW, lanes)
            _store1(pc, p * (2 * PCCAP) + 2 * k + 1, dst, lanes)
            pcnt_c[p] = k + 1
            return c2
        lax.fori_loop(0, nc_, bc, 0)
        return carry
    lax.fori_loop(0, NW, reb, 0)

    npan_full = jnp.where(o < 30, 64, jnp.where(o == 30, 33, 0))
    ng = e_dim // 16

    def fill_panel(tab, p, pan):
        cps = []
        for r in range(e_dim // 8):
            cps.append(pltpu.async_copy(
                tab.at[pl.ds(r * 8, 8), pl.ds(vbase + p * PW, PW)],
                pan.at[pl.ds(r * 8, 8), :], psemA))
        for cp in cps:
            cp.wait()

    def serve(g_hbm, pbuf, cap, row, dbuf, dump, cnt_ref, pan, ssem,
              defer):
        def inner(p, carry):
            if defer:
                @pl.when(p >= 2)
                def _():
                    pltpu.make_async_copy(row, g_hbm.at[dbuf.at[0]],
                                          ssem).wait()
            nslot = cap // 16
            for s in range(nslot):
                dbuf[0, pl.ds(s * 16, 16)] = dump + s * 16 + lanes
            n = cnt_ref[p]

            def one(i, c2):
                dv = _load1(pbuf, p * (2 * cap) + 2 * i)
                dst = _load1(pbuf, p * (2 * cap) + 2 * i + 1)
                for g in range(ng):
                    col = plsc.load_gather(
                        pan, [g * 16 + lanes, jnp.full((16,), dv,
                                                       jnp.int32)])
                    row[i, pl.ds(g * 16, 16)] = col
                _store1_2d(dbuf, i, dst, lanes)
                return c2
            lax.fori_loop(0, n, one, 0)
            cp = pltpu.async_copy(row, g_hbm.at[dbuf.at[0]], ssem)
            if not defer:
                cp.wait()
            return carry
        return inner

    def sweep_table(tab, g_hbm, pbuf, cap, rows, dbufs, dump, cnt_ref):
        srv = [serve(g_hbm, pbuf, cap, rows[b], dbufs[b], dump, cnt_ref,
                     panel, ssems[b], True) for b in (0, 1)]

        def two(p2, carry):
            p = p2 * 2

            @pl.when(p < npan_full)
            def _():
                fill_panel(tab, p, panel)
                srv[0](p, 0)

            @pl.when(p + 1 < npan_full)
            def _():
                fill_panel(tab, p + 1, panel)
                srv[1](p + 1, 0)
            return carry
        lax.fori_loop(0, 32, two, 0)
        for b in (0, 1):
            @pl.when(npan_full >= b + 1)
            def _():
                pltpu.make_async_copy(rows[b], g_hbm.at[dbufs[b].at[0]],
                                      ssems[b]).wait()

    sweep_table(ttab, te_g, pt, PTCAP, (trow, trow2), (tdst, tdst2),
                dump_t, pcnt_t)
    sweep_table(ctab, ce_g, pc, PCCAP, (crow, crow2), (cdst, cdst2),
                dump_c, pcnt_c)

    @pl.when(o == 30)
    def _tail():
        pltpu.sync_copy(ttail, tailp)
        serve(te_g, pt, PTCAP, trow, tdst, dump_t, pcnt_t, tailp,
              ssems[0], False)(33, 0)
        pltpu.sync_copy(ctail, tailp)
        serve(ce_g, pc, PCCAP, crow, cdst, dump_c, pcnt_c, tailp,
              ssems[1], False)(33, 0)


def _dots_body(te_hbm, ce_hbm, out_hbm, te_v, ce_v, out_v, sem,
               *, b_per_w, b_dim, c_dim, e_dim):
    wid = lax.axis_index("s") * NC + lax.axis_index("c")
    lanes = lax.iota(jnp.int32, 16)
    cb = 128
    for chunk in range(b_per_w // cb):
        base = wid * b_per_w + chunk * cb
        copies = [pltpu.async_copy(te_hbm.at[pl.ds(base, cb)], te_v, sem)]
        for c in range(c_dim):
            copies.append(pltpu.async_copy(
                ce_hbm.at[pl.ds(c * b_dim + base, cb)],
                ce_v.at[pl.ds(c * cb, cb)], sem))
        for cp in copies:
            cp.wait()

        for blk in range(cb // 16):
            rows16 = blk * 16 + lanes
            zero = jnp.zeros((16,), jnp.float32)

            def ebody(e, accs):
                ecol = jnp.full((16,), e, jnp.int32)
                tg = plsc.load_gather(te_v, [rows16, ecol])
                return tuple(
                    accs[c] + tg * plsc.load_gather(
                        ce_v, [c * cb + rows16, ecol])
                    for c in range(c_dim))

            accs = lax.fori_loop(0, e_dim, ebody, (zero,) * c_dim)
            for c in range(c_dim):
                out_v[c, pl.ds(blk * 16, 16)] = accs[c]

        pltpu.sync_copy(out_v, out_hbm.at[:, pl.ds(base, cb)])


def kernel(target, context, target_table, context_table):
    b_dim = target.shape[0]
    c_dim = context.shape[1]
    e_dim = target_table.shape[1]
    b_per_w = b_dim // NW

    ctx_t = context.T
    tt_t = target_table.T
    ct_t = context_table.T

    mesh = plsc.VectorSubcoreMesh(core_axis_name="c", subcore_axis_name="s")
    params = pltpu.CompilerParams(needs_layout_passes=False)

    bucketize = functools.partial(
        pl.kernel, mesh=mesh, compiler_params=params,
        out_type=(
            jax.ShapeDtypeStruct((NW, 32 * 2 * TCAP), jnp.int32),
            jax.ShapeDtypeStruct((NW, 32 * 2 * CCAP), jnp.int32),
            jax.ShapeDtypeStruct((NW * 8,), jnp.int32),
        ),
        scratch_types=[
            pltpu.VMEM((b_per_w,), jnp.int32),
            pltpu.VMEM((c_dim, b_per_w), jnp.int32),
            pltpu.VMEM((32 * 2 * TCAP,), jnp.int32),
            pltpu.VMEM((32 * 2 * CCAP,), jnp.int32),
            pltpu.SMEM((32,), jnp.int32),
            pltpu.SMEM((32,), jnp.int32),
            pltpu.SemaphoreType.DMA,
        ],
    )(functools.partial(_bucketize_body, b_per_w=b_per_w, c_dim=c_dim))
    tb, cb_, _sent = bucketize(target, ctx_t)

    sweep = functools.partial(
        pl.kernel, mesh=mesh, compiler_params=params,
        out_type=(
            jax.ShapeDtypeStruct((b_dim + PTCAP, 128), jnp.float32),
            jax.ShapeDtypeStruct((b_dim * c_dim + PCCAP, 128), jnp.float32),
        ),
        scratch_types=[
            pltpu.VMEM((e_dim, PW), jnp.float32),
            pltpu.VMEM((e_dim, 64), jnp.float32),
            pltpu.VMEM((32, 2 * TCAP), jnp.int32),
            pltpu.VMEM((32, 2 * CCAP), jnp.int32),
            pltpu.VMEM((64 * 2 * PTCAP,), jnp.int32),
            pltpu.VMEM((64 * 2 * PCCAP,), jnp.int32),
            pltpu.VMEM((PTCAP, 128), jnp.float32),
            pltpu.VMEM((PTCAP, 128), jnp.float32),
            pltpu.VMEM((PCCAP, 128), jnp.float32),
            pltpu.VMEM((PCCAP, 128), jnp.float32),
            pltpu.VMEM((1, PTCAP), jnp.int32),
            pltpu.VMEM((1, PTCAP), jnp.int32),
            pltpu.VMEM((1, PCCAP), jnp.int32),
            pltpu.VMEM((1, PCCAP), jnp.int32),
            pltpu.SMEM((64,), jnp.int32),
            pltpu.SMEM((64,), jnp.int32),
            pltpu.SemaphoreType.DMA,
            pltpu.SemaphoreType.DMA,
            pltpu.SemaphoreType.DMA,
            pltpu.SemaphoreType.DMA,
        ],
    )(functools.partial(_sweep_body, e_dim=e_dim, b_dim=b_dim, c_dim=c_dim))
    vfull = (SPAN * 30) + ((target_table.shape[0] - SPAN * 30) // PW) * PW
    tt_tail = target_table[vfull:, :].T
    ct_tail = context_table[vfull:, :].T
    te_g, ce_g = sweep(tt_t, ct_t, tt_tail, ct_tail, tb, cb_)

    dots = functools.partial(
        pl.kernel, mesh=mesh, compiler_params=params,
        out_type=jax.ShapeDtypeStruct((c_dim, b_dim), jnp.float32),
        scratch_types=[
            pltpu.VMEM((128, 128), jnp.float32),
            pltpu.VMEM((c_dim * 128, 128), jnp.float32),
            pltpu.VMEM((c_dim, 128), jnp.float32),
            pltpu.SemaphoreType.DMA,
        ],
    )(functools.partial(_dots_body, b_per_w=b_per_w, b_dim=b_dim,
                        c_dim=c_dim, e_dim=e_dim))
    out = dots(te_g, ce_g)
    return out.T


# speedup vs baseline: 7.6676x; 1.0819x over previous
import functools

import jax
import jax.numpy as jnp
from jax import lax
from jax.experimental import pallas as pl
from jax.experimental.pallas import tpu as pltpu
from jax.experimental.pallas import tpu_sc as plsc

NC = 2
NS = 16
NW = NC * NS

OWN_SHIFT = 15
SPAN = 1 << OWN_SHIFT
PW = 512
TCAP = 64
CCAP = 192
PTCAP = 32
PCCAP = 96


def _lane0(lanes):
    return lanes == 0


def _store1(ref, pos, val, lanes):
    plsc.store_scatter(ref, [jnp.full((16,), pos, jnp.int32)],
                       jnp.full((16,), val, ref.dtype), mask=_lane0(lanes))


def _load1(ref, *pos):
    idx = [jnp.full((16,), p, jnp.int32) for p in pos]
    return plsc.load_gather(ref, idx)[0]


def _store1_2d(ref, col, val, lanes):
    plsc.store_scatter(ref,
                       [jnp.zeros((16,), jnp.int32),
                        jnp.full((16,), col, jnp.int32)],
                       jnp.full((16,), val, ref.dtype), mask=_lane0(lanes))


def _bucketize_body(tgt_hbm, ctx_hbm, tb_hbm, cb_hbm, out_hbm,
                    tv, cv, st_t, st_c, cnt_t, cnt_c, sem,
                    *, b_per_w, c_dim):
    wid = lax.axis_index("s") * NC + lax.axis_index("c")
    base = wid * b_per_w
    lanes = lax.iota(jnp.int32, 16)
    pltpu.sync_copy(tgt_hbm.at[pl.ds(base, b_per_w)], tv)
    pltpu.sync_copy(ctx_hbm.at[:, pl.ds(base, b_per_w)], cv)

    def zero_cnt(i, carry):
        cnt_t[i] = 0
        cnt_c[i] = 0
        return carry
    lax.fori_loop(0, 32, zero_cnt, 0)

    def req(i, carry):
        v = _load1(tv, i)
        o = jnp.minimum(lax.shift_right_logical(v, OWN_SHIFT), 30)
        k = cnt_t[o]
        _store1(st_t, o * (2 * TCAP) + 2 * k, v, lanes)
        _store1(st_t, o * (2 * TCAP) + 2 * k + 1, base + i, lanes)
        cnt_t[o] = k + 1
        for c in range(c_dim):
            v2 = _load1(cv, c, i)
            o2 = jnp.minimum(lax.shift_right_logical(v2, OWN_SHIFT), 30)
            k2 = cnt_c[o2]
            _store1(st_c, o2 * (2 * CCAP) + 2 * k2, v2, lanes)
            _store1(st_c, o2 * (2 * CCAP) + 2 * k2 + 1,
                    c * (b_per_w * NW) + base + i, lanes)
            cnt_c[o2] = k2 + 1
        return carry
    lax.fori_loop(0, b_per_w, req, 0)

    def pub(o, carry):
        _store1(st_t, o * (2 * TCAP) + 2 * TCAP - 2, cnt_t[o], lanes)
        _store1(st_c, o * (2 * CCAP) + 2 * CCAP - 2, cnt_c[o], lanes)
        return carry
    lax.fori_loop(0, 32, pub, 0)

    pltpu.sync_copy(st_t, tb_hbm.at[wid])
    pltpu.sync_copy(st_c, cb_hbm.at[wid])
    out_v = tv
    pltpu.sync_copy(out_v.at[pl.ds(0, 8)], out_hbm.at[pl.ds(wid * 8, 8)])


def _sweep_body(ttab, ctab, ttail, ctail, tb_hbm, cb_hbm, te_g, ce_g,
                panel, tailp, rq_t, rq_c, pt, pc, trow, trow2, crow,
                crow2, tdst, tdst2, cdst, cdst2, pcnt_t, pcnt_c,
                sem, psemA, ssemA, ssemB, *, e_dim, b_dim, c_dim):
    ssems = (ssemA, ssemB)
    o = lax.axis_index("s") * NC + lax.axis_index("c")
    lanes = lax.iota(jnp.int32, 16)
    vbase = o * SPAN
    dump_t = jnp.int32(b_dim)
    dump_c = jnp.int32(b_dim * c_dim)

    pltpu.sync_copy(tb_hbm.at[:, pl.ds(o * (2 * TCAP), 2 * TCAP)], rq_t)
    pltpu.sync_copy(cb_hbm.at[:, pl.ds(o * (2 * CCAP), 2 * CCAP)], rq_c)

    def zc(i, carry):
        pcnt_t[i] = 0
        pcnt_c[i] = 0
        return carry
    lax.fori_loop(0, 64, zc, 0)

    def reb(w, carry):
        nt = _load1(rq_t, w, 2 * TCAP - 2)
        nc_ = _load1(rq_c, w, 2 * CCAP - 2)

        def bt(i, c2):
            v = _load1(rq_t, w, 2 * i)
            dst = _load1(rq_t, w, 2 * i + 1)
            dv = v - vbase
            p = lax.shift_right_logical(dv, 9)
            k = pcnt_t[p]
            _store1(pt, p * (2 * PTCAP) + 2 * k, dv - p * PW, lanes)
            _store1(pt, p * (2 * PTCAP) + 2 * k + 1, dst, lanes)
            pcnt_t[p] = k + 1
            return c2
        lax.fori_loop(0, nt, bt, 0)

        def bc(i, c2):
            v = _load1(rq_c, w, 2 * i)
            dst = _load1(rq_c, w, 2 * i + 1)
            dv = v - vbase
            p = lax.shift_right_logical(dv, 9)
            k = pcnt_c[p]
            _store1(pc, p * (2 * PCCAP) + 2 * k, dv - p * PW, lanes)
            _store1(pc, p * (2 * PCCAP) + 2 * k + 1, dst, lanes)
            pcnt_c[p] = k + 1
            return c2
        lax.fori_loop(0, nc_, bc, 0)
        return carry
    lax.fori_loop(0, NW, reb, 0)

    npan_full = jnp.where(o < 30, 64, jnp.where(o == 30, 33, 0))
    ng = e_dim // 16

    def fill_panel(tab, p, pan):
        cps = []
        for r in range(e_dim // 8):
            cps.append(pltpu.async_copy(
                tab.at[pl.ds(r * 8, 8), pl.ds(vbase + p * PW, PW)],
                pan.at[pl.ds(r * 8, 8), :], psemA))
        for cp in cps:
            cp.wait()

    def serve(g_hbm, pbuf, cap, row, dbuf, dump, cnt_ref, pan, ssem,
              defer):
        def inner(p, carry):
            if defer:
                @pl.when(p >= 2)
                def _():
                    pltpu.make_async_copy(row, g_hbm.at[dbuf.at[0]],
                                          ssem).wait()
            nslot = cap // 16
            for s in range(nslot):
                dbuf[0, pl.ds(s * 16, 16)] = dump + s * 16 + lanes
            n = cnt_ref[p]

            def one(i, c2):
                dv = _load1(pbuf, p * (2 * cap) + 2 * i)
                dst = _load1(pbuf, p * (2 * cap) + 2 * i + 1)
                for g in range(ng):
                    col = plsc.load_gather(
                        pan, [g * 16 + lanes, jnp.full((16,), dv,
                                                       jnp.int32)])
                    row[i, pl.ds(g * 16, 16)] = col
                _store1_2d(dbuf, i, dst, lanes)
                return c2
            lax.fori_loop(0, n, one, 0)
            cp = pltpu.async_copy(row, g_hbm.at[dbuf.at[0]], ssem)
            if not defer:
                cp.wait()
            return carry
        return inner

    def sweep_table(tab, g_hbm, pbuf, cap, rows, dbufs, dump, cnt_ref):
        srv = [serve(g_hbm, pbuf, cap, rows[b], dbufs[b], dump, cnt_ref,
                     panel, ssems[b], True) for b in (0, 1)]

        def two(p2, carry):
            p = p2 * 2

            @pl.when(p < npan_full)
            def _():
                fill_panel(tab, p, panel)
                srv[0](p, 0)

            @pl.when(p + 1 < npan_full)
            def _():
                fill_panel(tab, p + 1, panel)
                srv[1](p + 1, 0)
            return carry
        lax.fori_loop(0, 32, two, 0)
        for b in (0, 1):
            @pl.when(npan_full >= b + 1)
            def _():
                pltpu.make_async_copy(rows[b], g_hbm.at[dbufs[b].at[0]],
                                      ssems[b]).wait()

    sweep_table(ttab, te_g, pt, PTCAP, (trow, trow2), (tdst, tdst2),
                dump_t, pcnt_t)
    sweep_table(ctab, ce_g, pc, PCCAP, (crow, crow2), (cdst, cdst2),
                dump_c, pcnt_c)

    @pl.when(o == 30)
    def _tail():
        pltpu.sync_copy(ttail, tailp)
        serve(te_g, pt, PTCAP, trow, tdst, dump_t, pcnt_t, tailp,
              ssems[0], False)(33, 0)
        pltpu.sync_copy(ctail, tailp)
        serve(ce_g, pc, PCCAP, crow, cdst, dump_c, pcnt_c, tailp,
              ssems[1], False)(33, 0)


def _dots_body(te_hbm, ce_hbm, out_hbm, te_v, ce_v, out_v, sem,
               *, b_per_w, b_dim, c_dim, e_dim):
    wid = lax.axis_index("s") * NC + lax.axis_index("c")
    lanes = lax.iota(jnp.int32, 16)
    cb = 128
    for chunk in range(b_per_w // cb):
        base = wid * b_per_w + chunk * cb
        copies = [pltpu.async_copy(te_hbm.at[pl.ds(base, cb)], te_v, sem)]
        for c in range(c_dim):
            copies.append(pltpu.async_copy(
                ce_hbm.at[pl.ds(c * b_dim + base, cb)],
                ce_v.at[pl.ds(c * cb, cb)], sem))
        for cp in copies:
            cp.wait()

        for blk in range(cb // 16):
            rows16 = blk * 16 + lanes
            zero = jnp.zeros((16,), jnp.float32)

            def ebody(e4, accs):
                for u in range(4):
                    e = e4 * 4 + u
                    ecol = jnp.full((16,), e, jnp.int32)
                    tg = plsc.load_gather(te_v, [rows16, ecol])
                    accs = tuple(
                        accs[c] + tg * plsc.load_gather(
                            ce_v, [c * cb + rows16, ecol])
                        for c in range(c_dim))
                return accs

            accs = lax.fori_loop(0, e_dim // 4, ebody, (zero,) * c_dim)
            for c in range(c_dim):
                out_v[c, pl.ds(blk * 16, 16)] = accs[c]

        pltpu.sync_copy(out_v, out_hbm.at[:, pl.ds(base, cb)])


def kernel(target, context, target_table, context_table):
    b_dim = target.shape[0]
    c_dim = context.shape[1]
    e_dim = target_table.shape[1]
    b_per_w = b_dim // NW

    ctx_t = context.T
    tt_t = target_table.T
    ct_t = context_table.T

    mesh = plsc.VectorSubcoreMesh(core_axis_name="c", subcore_axis_name="s")
    params = pltpu.CompilerParams(needs_layout_passes=False)

    bucketize = functools.partial(
        pl.kernel, mesh=mesh, compiler_params=params,
        out_type=(
            jax.ShapeDtypeStruct((NW, 32 * 2 * TCAP), jnp.int32),
            jax.ShapeDtypeStruct((NW, 32 * 2 * CCAP), jnp.int32),
            jax.ShapeDtypeStruct((NW * 8,), jnp.int32),
        ),
        scratch_types=[
            pltpu.VMEM((b_per_w,), jnp.int32),
            pltpu.VMEM((c_dim, b_per_w), jnp.int32),
            pltpu.VMEM((32 * 2 * TCAP,), jnp.int32),
            pltpu.VMEM((32 * 2 * CCAP,), jnp.int32),
            pltpu.SMEM((32,), jnp.int32),
            pltpu.SMEM((32,), jnp.int32),
            pltpu.SemaphoreType.DMA,
        ],
    )(functools.partial(_bucketize_body, b_per_w=b_per_w, c_dim=c_dim))
    tb, cb_, _sent = bucketize(target, ctx_t)

    sweep = functools.partial(
        pl.kernel, mesh=mesh, compiler_params=params,
        out_type=(
            jax.ShapeDtypeStruct((b_dim + PTCAP, 128), jnp.float32),
            jax.ShapeDtypeStruct((b_dim * c_dim + PCCAP, 128), jnp.float32),
        ),
        scratch_types=[
            pltpu.VMEM((e_dim, PW), jnp.float32),
            pltpu.VMEM((e_dim, 64), jnp.float32),
            pltpu.VMEM((32, 2 * TCAP), jnp.int32),
            pltpu.VMEM((32, 2 * CCAP), jnp.int32),
            pltpu.VMEM((64 * 2 * PTCAP,), jnp.int32),
            pltpu.VMEM((64 * 2 * PCCAP,), jnp.int32),
            pltpu.VMEM((PTCAP, 128), jnp.float32),
            pltpu.VMEM((PTCAP, 128), jnp.float32),
            pltpu.VMEM((PCCAP, 128), jnp.float32),
            pltpu.VMEM((PCCAP, 128), jnp.float32),
            pltpu.VMEM((1, PTCAP), jnp.int32),
            pltpu.VMEM((1, PTCAP), jnp.int32),
            pltpu.VMEM((1, PCCAP), jnp.int32),
            pltpu.VMEM((1, PCCAP), jnp.int32),
            pltpu.SMEM((64,), jnp.int32),
            pltpu.SMEM((64,), jnp.int32),
            pltpu.SemaphoreType.DMA,
            pltpu.SemaphoreType.DMA,
            pltpu.SemaphoreType.DMA,
            pltpu.SemaphoreType.DMA,
        ],
    )(functools.partial(_sweep_body, e_dim=e_dim, b_dim=b_dim, c_dim=c_dim))
    vfull = (SPAN * 30) + ((target_table.shape[0] - SPAN * 30) // PW) * PW
    tt_tail = target_table[vfull:, :].T
    ct_tail = context_table[vfull:, :].T
    te_g, ce_g = sweep(tt_t, ct_t, tt_tail, ct_tail, tb, cb_)

    dots = functools.partial(
        pl.kernel, mesh=mesh, compiler_params=params,
        out_type=jax.ShapeDtypeStruct((c_dim, b_dim), jnp.float32),
        scratch_types=[
            pltpu.VMEM((128, 128), jnp.float32),
            pltpu.VMEM((c_dim * 128, 128), jnp.float32),
            pltpu.VMEM((c_dim, 128), jnp.float32),
            pltpu.SemaphoreType.DMA,
        ],
    )(functools.partial(_dots_body, b_per_w=b_per_w, b_dim=b_dim,
                        c_dim=c_dim, e_dim=e_dim))
    out = dots(te_g, ce_g)
    return out.T


# speedup vs baseline: 8.9075x; 1.1617x over previous
import functools

import jax
import jax.numpy as jnp
from jax import lax
from jax.experimental import pallas as pl
from jax.experimental.pallas import tpu as pltpu
from jax.experimental.pallas import tpu_sc as plsc

NC = 2
NS = 16
NW = NC * NS

OWN_SHIFT = 15
SPAN = 1 << OWN_SHIFT
PW = 512
TCAP = 64
CCAP = 192
PTCAP = 32
PCCAP = 96


def _lane0(lanes):
    return lanes == 0


def _store1(ref, pos, val, lanes):
    plsc.store_scatter(ref, [jnp.full((16,), pos, jnp.int32)],
                       jnp.full((16,), val, ref.dtype), mask=_lane0(lanes))


def _load1(ref, *pos):
    idx = [jnp.full((16,), p, jnp.int32) for p in pos]
    return plsc.load_gather(ref, idx)[0]


def _store1_2d(ref, col, val, lanes):
    plsc.store_scatter(ref,
                       [jnp.zeros((16,), jnp.int32),
                        jnp.full((16,), col, jnp.int32)],
                       jnp.full((16,), val, ref.dtype), mask=_lane0(lanes))


def _bucketize_body(tgt_hbm, ctx_hbm, tb_hbm, cb_hbm, out_hbm,
                    tv, cv, st_t, st_c, cnt_t, cnt_c, sem,
                    *, b_per_w, c_dim):
    wid = lax.axis_index("s") * NC + lax.axis_index("c")
    base = wid * b_per_w
    lanes = lax.iota(jnp.int32, 16)
    pltpu.sync_copy(tgt_hbm.at[pl.ds(base, b_per_w)], tv)
    pltpu.sync_copy(ctx_hbm.at[:, pl.ds(base, b_per_w)], cv)

    def zero_cnt(i, carry):
        cnt_t[i] = 0
        cnt_c[i] = 0
        return carry
    lax.fori_loop(0, 32, zero_cnt, 0)

    def req16(j, carry):
        i0 = j * 16
        tvv = tv[pl.ds(i0, 16)]
        ovv = jnp.minimum(lax.shift_right_logical(tvv, OWN_SHIFT), 30)
        for l in range(16):
            v, o = tvv[l], ovv[l]
            k = cnt_t[o]
            _store1(st_t, o * (2 * TCAP) + 2 * k, v, lanes)
            _store1(st_t, o * (2 * TCAP) + 2 * k + 1, base + i0 + l, lanes)
            cnt_t[o] = k + 1
        for c in range(c_dim):
            cvv = cv[c, pl.ds(i0, 16)]
            ov2 = jnp.minimum(lax.shift_right_logical(cvv, OWN_SHIFT), 30)
            for l in range(16):
                v2, o2 = cvv[l], ov2[l]
                k2 = cnt_c[o2]
                _store1(st_c, o2 * (2 * CCAP) + 2 * k2, v2, lanes)
                _store1(st_c, o2 * (2 * CCAP) + 2 * k2 + 1,
                        c * (b_per_w * NW) + base + i0 + l, lanes)
                cnt_c[o2] = k2 + 1
        return carry
    lax.fori_loop(0, b_per_w // 16, req16, 0)

    def pub(o, carry):
        _store1(st_t, o * (2 * TCAP) + 2 * TCAP - 2, cnt_t[o], lanes)
        _store1(st_c, o * (2 * CCAP) + 2 * CCAP - 2, cnt_c[o], lanes)
        return carry
    lax.fori_loop(0, 32, pub, 0)

    pltpu.sync_copy(st_t, tb_hbm.at[wid])
    pltpu.sync_copy(st_c, cb_hbm.at[wid])
    out_v = tv
    pltpu.sync_copy(out_v.at[pl.ds(0, 8)], out_hbm.at[pl.ds(wid * 8, 8)])


def _sweep_body(ttab, ctab, ttail, ctail, tb_hbm, cb_hbm, te_g, ce_g,
                panel, tailp, rq_t, rq_c, pt, pc, trow, trow2, crow,
                crow2, tdst, tdst2, cdst, cdst2, pcnt_t, pcnt_c,
                sem, psemA, ssemA, ssemB, *, e_dim, b_dim, c_dim):
    ssems = (ssemA, ssemB)
    o = lax.axis_index("s") * NC + lax.axis_index("c")
    lanes = lax.iota(jnp.int32, 16)
    vbase = o * SPAN
    dump_t = jnp.int32(b_dim)
    dump_c = jnp.int32(b_dim * c_dim)

    pltpu.sync_copy(tb_hbm.at[:, pl.ds(o * (2 * TCAP), 2 * TCAP)], rq_t)
    pltpu.sync_copy(cb_hbm.at[:, pl.ds(o * (2 * CCAP), 2 * CCAP)], rq_c)

    def zc(i, carry):
        pcnt_t[i] = 0
        pcnt_c[i] = 0
        return carry
    lax.fori_loop(0, 65, zc, 0)

    def reb(w, carry):
        nt = _load1(rq_t, w, 2 * TCAP - 2)
        nc_ = _load1(rq_c, w, 2 * CCAP - 2)
        wv = jnp.full((16,), w, jnp.int32)

        def mk(rq, n, pbuf, cap, pcnt):
            def b16(j, c2):
                ii = j * 16 + lanes
                vv = plsc.load_gather(rq, [wv, 2 * ii])
                dd = plsc.load_gather(rq, [wv, 2 * ii + 1])
                dvv = vv - vbase
                pv = lax.shift_right_logical(dvv, 9)
                pv = jnp.where(ii < n, pv, 64)
                colv = dvv - pv * PW
                lim = 65 * 2 * cap - 2
                for l in range(16):
                    p, dv, dst = pv[l], colv[l], dd[l]
                    k = pcnt[p]
                    pos = jnp.minimum(p * (2 * cap) + 2 * k, lim)
                    _store1(pbuf, pos, dv, lanes)
                    _store1(pbuf, pos + 1, dst, lanes)
                    pcnt[p] = k + 1
                return c2
            lax.fori_loop(0, (n + 15) // 16, b16, 0)

        mk(rq_t, nt, pt, PTCAP, pcnt_t)
        mk(rq_c, nc_, pc, PCCAP, pcnt_c)
        return carry
    lax.fori_loop(0, NW, reb, 0)

    npan_full = jnp.where(o < 30, 64, jnp.where(o == 30, 33, 0))
    ng = e_dim // 16

    def fill_panel(tab, p, pan):
        cps = []
        for r in range(e_dim // 8):
            cps.append(pltpu.async_copy(
                tab.at[pl.ds(r * 8, 8), pl.ds(vbase + p * PW, PW)],
                pan.at[pl.ds(r * 8, 8), :], psemA))
        for cp in cps:
            cp.wait()

    def serve(g_hbm, pbuf, cap, row, dbuf, dump, cnt_ref, pan, ssem,
              defer):
        def inner(p, carry):
            if defer:
                @pl.when(p >= 2)
                def _():
                    pltpu.make_async_copy(row, g_hbm.at[dbuf.at[0]],
                                          ssem).wait()
            nslot = cap // 16
            for s in range(nslot):
                dbuf[0, pl.ds(s * 16, 16)] = dump + s * 16 + lanes
            n = cnt_ref[p]

            def one(i, c2):
                dv = _load1(pbuf, p * (2 * cap) + 2 * i)
                dst = _load1(pbuf, p * (2 * cap) + 2 * i + 1)
                for g in range(ng):
                    col = plsc.load_gather(
                        pan, [g * 16 + lanes, jnp.full((16,), dv,
                                                       jnp.int32)])
                    row[i, pl.ds(g * 16, 16)] = col
                _store1_2d(dbuf, i, dst, lanes)
                return c2
            lax.fori_loop(0, n, one, 0)
            cp = pltpu.async_copy(row, g_hbm.at[dbuf.at[0]], ssem)
            if not defer:
                cp.wait()
            return carry
        return inner

    def sweep_table(tab, g_hbm, pbuf, cap, rows, dbufs, dump, cnt_ref):
        srv = [serve(g_hbm, pbuf, cap, rows[b], dbufs[b], dump, cnt_ref,
                     panel, ssems[b], True) for b in (0, 1)]

        def two(p2, carry):
            p = p2 * 2

            @pl.when(p < npan_full)
            def _():
                fill_panel(tab, p, panel)
                srv[0](p, 0)

            @pl.when(p + 1 < npan_full)
            def _():
                fill_panel(tab, p + 1, panel)
                srv[1](p + 1, 0)
            return carry
        lax.fori_loop(0, 32, two, 0)
        for b in (0, 1):
            @pl.when(npan_full >= b + 1)
            def _():
                pltpu.make_async_copy(rows[b], g_hbm.at[dbufs[b].at[0]],
                                      ssems[b]).wait()

    sweep_table(ttab, te_g, pt, PTCAP, (trow, trow2), (tdst, tdst2),
                dump_t, pcnt_t)
    sweep_table(ctab, ce_g, pc, PCCAP, (crow, crow2), (cdst, cdst2),
                dump_c, pcnt_c)

    @pl.when(o == 30)
    def _tail():
        pltpu.sync_copy(ttail, tailp)
        serve(te_g, pt, PTCAP, trow, tdst, dump_t, pcnt_t, tailp,
              ssems[0], False)(33, 0)
        pltpu.sync_copy(ctail, tailp)
        serve(ce_g, pc, PCCAP, crow, cdst, dump_c, pcnt_c, tailp,
              ssems[1], False)(33, 0)


def _dots_body(te_hbm, ce_hbm, out_hbm, te_v, ce_v, out_v, sem,
               *, b_per_w, b_dim, c_dim, e_dim):
    wid = lax.axis_index("s") * NC + lax.axis_index("c")
    lanes = lax.iota(jnp.int32, 16)
    cb = 128
    for chunk in range(b_per_w // cb):
        base = wid * b_per_w + chunk * cb
        copies = [pltpu.async_copy(te_hbm.at[pl.ds(base, cb)], te_v, sem)]
        for c in range(c_dim):
            copies.append(pltpu.async_copy(
                ce_hbm.at[pl.ds(c * b_dim + base, cb)],
                ce_v.at[pl.ds(c * cb, cb)], sem))
        for cp in copies:
            cp.wait()

        for blk in range(cb // 16):
            rows16 = blk * 16 + lanes
            zero = jnp.zeros((16,), jnp.float32)

            def ebody(e4, accs):
                for u in range(4):
                    e = e4 * 4 + u
                    ecol = jnp.full((16,), e, jnp.int32)
                    tg = plsc.load_gather(te_v, [rows16, ecol])
                    accs = tuple(
                        accs[c] + tg * plsc.load_gather(
                            ce_v, [c * cb + rows16, ecol])
                        for c in range(c_dim))
                return accs

            accs = lax.fori_loop(0, e_dim // 4, ebody, (zero,) * c_dim)
            for c in range(c_dim):
                out_v[c, pl.ds(blk * 16, 16)] = accs[c]

        pltpu.sync_copy(out_v, out_hbm.at[:, pl.ds(base, cb)])


def kernel(target, context, target_table, context_table):
    b_dim = target.shape[0]
    c_dim = context.shape[1]
    e_dim = target_table.shape[1]
    b_per_w = b_dim // NW

    ctx_t = context.T
    tt_t = target_table.T
    ct_t = context_table.T

    mesh = plsc.VectorSubcoreMesh(core_axis_name="c", subcore_axis_name="s")
    params = pltpu.CompilerParams(needs_layout_passes=False)

    bucketize = functools.partial(
        pl.kernel, mesh=mesh, compiler_params=params,
        out_type=(
            jax.ShapeDtypeStruct((NW, 32 * 2 * TCAP), jnp.int32),
            jax.ShapeDtypeStruct((NW, 32 * 2 * CCAP), jnp.int32),
            jax.ShapeDtypeStruct((NW * 8,), jnp.int32),
        ),
        scratch_types=[
            pltpu.VMEM((b_per_w,), jnp.int32),
            pltpu.VMEM((c_dim, b_per_w), jnp.int32),
            pltpu.VMEM((32 * 2 * TCAP,), jnp.int32),
            pltpu.VMEM((32 * 2 * CCAP,), jnp.int32),
            pltpu.SMEM((32,), jnp.int32),
            pltpu.SMEM((32,), jnp.int32),
            pltpu.SemaphoreType.DMA,
        ],
    )(functools.partial(_bucketize_body, b_per_w=b_per_w, c_dim=c_dim))
    tb, cb_, _sent = bucketize(target, ctx_t)

    sweep = functools.partial(
        pl.kernel, mesh=mesh, compiler_params=params,
        out_type=(
            jax.ShapeDtypeStruct((b_dim + PTCAP, 128), jnp.float32),
            jax.ShapeDtypeStruct((b_dim * c_dim + PCCAP, 128), jnp.float32),
        ),
        scratch_types=[
            pltpu.VMEM((e_dim, PW), jnp.float32),
            pltpu.VMEM((e_dim, 64), jnp.float32),
            pltpu.VMEM((32, 2 * TCAP), jnp.int32),
            pltpu.VMEM((32, 2 * CCAP), jnp.int32),
            pltpu.VMEM((65 * 2 * PTCAP,), jnp.int32),
            pltpu.VMEM((65 * 2 * PCCAP,), jnp.int32),
            pltpu.VMEM((PTCAP, 128), jnp.float32),
            pltpu.VMEM((PTCAP, 128), jnp.float32),
            pltpu.VMEM((PCCAP, 128), jnp.float32),
            pltpu.VMEM((PCCAP, 128), jnp.float32),
            pltpu.VMEM((1, PTCAP), jnp.int32),
            pltpu.VMEM((1, PTCAP), jnp.int32),
            pltpu.VMEM((1, PCCAP), jnp.int32),
            pltpu.VMEM((1, PCCAP), jnp.int32),
            pltpu.SMEM((65,), jnp.int32),
            pltpu.SMEM((65,), jnp.int32),
            pltpu.SemaphoreType.DMA,
            pltpu.SemaphoreType.DMA,
            pltpu.SemaphoreType.DMA,
            pltpu.SemaphoreType.DMA,
        ],
    )(functools.partial(_sweep_body, e_dim=e_dim, b_dim=b_dim, c_dim=c_dim))
    vfull = (SPAN * 30) + ((target_table.shape[0] - SPAN * 30) // PW) * PW
    tt_tail = target_table[vfull:, :].T
    ct_tail = context_table[vfull:, :].T
    te_g, ce_g = sweep(tt_t, ct_t, tt_tail, ct_tail, tb, cb_)

    dots = functools.partial(
        pl.kernel, mesh=mesh, compiler_params=params,
        out_type=jax.ShapeDtypeStruct((c_dim, b_dim), jnp.float32),
        scratch_types=[
            pltpu.VMEM((128, 128), jnp.float32),
            pltpu.VMEM((c_dim * 128, 128), jnp.float32),
            pltpu.VMEM((c_dim, 128), jnp.float32),
            pltpu.SemaphoreType.DMA,
        ],
    )(functools.partial(_dots_body, b_per_w=b_per_w, b_dim=b_dim,
                        c_dim=c_dim, e_dim=e_dim))
    out = dots(te_g, ce_g)
    return out.T


# speedup vs baseline: 9.2698x; 1.0407x over previous
import functools

import jax
import jax.numpy as jnp
from jax import lax
from jax.experimental import pallas as pl
from jax.experimental.pallas import tpu as pltpu
from jax.experimental.pallas import tpu_sc as plsc

NC = 2
NS = 16
NW = NC * NS

OWN_SHIFT = 15
SPAN = 1 << OWN_SHIFT
PW = 512
TCAP = 64
CCAP = 192
PTCAP = 32
PCCAP = 96


def _lane0(lanes):
    return lanes == 0


def _store1(ref, pos, val, lanes):
    plsc.store_scatter(ref, [jnp.full((16,), pos, jnp.int32)],
                       jnp.full((16,), val, ref.dtype), mask=_lane0(lanes))


def _load1(ref, *pos):
    idx = [jnp.full((16,), p, jnp.int32) for p in pos]
    return plsc.load_gather(ref, idx)[0]


def _store1_2d(ref, col, val, lanes):
    plsc.store_scatter(ref,
                       [jnp.zeros((16,), jnp.int32),
                        jnp.full((16,), col, jnp.int32)],
                       jnp.full((16,), val, ref.dtype), mask=_lane0(lanes))


def _bucketize_body(tgt_hbm, ctx_hbm, tb_hbm, cb_hbm, out_hbm,
                    tv, cv, st_t, st_c, cnt_t, cnt_c, sem,
                    *, b_per_w, c_dim):
    wid = lax.axis_index("s") * NC + lax.axis_index("c")
    base = wid * b_per_w
    lanes = lax.iota(jnp.int32, 16)
    pltpu.sync_copy(tgt_hbm.at[pl.ds(base, b_per_w)], tv)
    pltpu.sync_copy(ctx_hbm.at[:, pl.ds(base, b_per_w)], cv)

    def zero_cnt(i, carry):
        cnt_t[i] = 0
        cnt_c[i] = 0
        return carry
    lax.fori_loop(0, 32, zero_cnt, 0)

    def req16(j, carry):
        i0 = j * 16
        tvv = tv[pl.ds(i0, 16)]
        ovv = jnp.minimum(lax.shift_right_logical(tvv, OWN_SHIFT), 30)
        for l in range(16):
            v, o = tvv[l], ovv[l]
            k = cnt_t[o]
            _store1(st_t, o * (2 * TCAP) + 2 * k, v, lanes)
            _store1(st_t, o * (2 * TCAP) + 2 * k + 1, base + i0 + l, lanes)
            cnt_t[o] = k + 1
        for c in range(c_dim):
            cvv = cv[c, pl.ds(i0, 16)]
            ov2 = jnp.minimum(lax.shift_right_logical(cvv, OWN_SHIFT), 30)
            for l in range(16):
                v2, o2 = cvv[l], ov2[l]
                k2 = cnt_c[o2]
                _store1(st_c, o2 * (2 * CCAP) + 2 * k2, v2, lanes)
                _store1(st_c, o2 * (2 * CCAP) + 2 * k2 + 1,
                        c * (b_per_w * NW) + base + i0 + l, lanes)
                cnt_c[o2] = k2 + 1
        return carry
    lax.fori_loop(0, b_per_w // 16, req16, 0)

    def pub(o, carry):
        _store1(st_t, o * (2 * TCAP) + 2 * TCAP - 2, cnt_t[o], lanes)
        _store1(st_c, o * (2 * CCAP) + 2 * CCAP - 2, cnt_c[o], lanes)
        return carry
    lax.fori_loop(0, 32, pub, 0)

    pltpu.sync_copy(st_t, tb_hbm.at[wid])
    pltpu.sync_copy(st_c, cb_hbm.at[wid])
    out_v = tv
    pltpu.sync_copy(out_v.at[pl.ds(0, 8)], out_hbm.at[pl.ds(wid * 8, 8)])


def _sweep_body(ttab, ctab, ttail, ctail, tb_hbm, cb_hbm, te_g, ce_g,
                panel, panel2, tailp, rq_t, rq_c, pt, pc, trow, crow,
                tdst, cdst, pcnt_t, pcnt_c,
                sem, psemA, psemB, ssemA, *, e_dim, b_dim, c_dim):
    o = lax.axis_index("s") * NC + lax.axis_index("c")
    lanes = lax.iota(jnp.int32, 16)
    vbase = o * SPAN
    dump_t = jnp.int32(b_dim)
    dump_c = jnp.int32(b_dim * c_dim)

    pltpu.sync_copy(tb_hbm.at[:, pl.ds(o * (2 * TCAP), 2 * TCAP)], rq_t)
    pltpu.sync_copy(cb_hbm.at[:, pl.ds(o * (2 * CCAP), 2 * CCAP)], rq_c)

    def zc(i, carry):
        pcnt_t[i] = 0
        pcnt_c[i] = 0
        return carry
    lax.fori_loop(0, 65, zc, 0)

    def reb(w, carry):
        nt = _load1(rq_t, w, 2 * TCAP - 2)
        nc_ = _load1(rq_c, w, 2 * CCAP - 2)
        wv = jnp.full((16,), w, jnp.int32)

        def mk(rq, n, pbuf, cap, pcnt):
            def b16(j, c2):
                ii = j * 16 + lanes
                vv = plsc.load_gather(rq, [wv, 2 * ii])
                dd = plsc.load_gather(rq, [wv, 2 * ii + 1])
                dvv = vv - vbase
                pv = lax.shift_right_logical(dvv, 9)
                pv = jnp.where(ii < n, pv, 64)
                colv = dvv - pv * PW
                packed = dd * PW + colv
                lim = 65 * cap - 1
                for l in range(16):
                    p, pk = pv[l], packed[l]
                    k = pcnt[p]
                    pos = jnp.minimum(p * cap + k, lim)
                    _store1(pbuf, pos, pk, lanes)
                    pcnt[p] = k + 1
                return c2
            lax.fori_loop(0, (n + 15) // 16, b16, 0)

        mk(rq_t, nt, pt, PTCAP, pcnt_t)
        mk(rq_c, nc_, pc, PCCAP, pcnt_c)
        return carry
    lax.fori_loop(0, NW, reb, 0)

    npan_full = jnp.where(o < 30, 64, jnp.where(o == 30, 33, 0))
    ng = e_dim // 16

    def fill_panel(tab, p, pan):
        cps = []
        for r in range(e_dim // 8):
            cps.append(pltpu.async_copy(
                tab.at[pl.ds(r * 8, 8), pl.ds(vbase + p * PW, PW)],
                pan.at[pl.ds(r * 8, 8), :], psemA))
        for cp in cps:
            cp.wait()

    def serve(g_hbm, pbuf, cap, row, dbuf, dump, cnt_ref, pan, defer):
        def inner(p, carry):
            if defer:
                @pl.when(p >= 1)
                def _():
                    pltpu.make_async_copy(row, g_hbm.at[dbuf.at[0]],
                                          ssemA).wait()
            nslot = cap // 16
            for s in range(nslot):
                dbuf[0, pl.ds(s * 16, 16)] = dump + s * 16 + lanes
            n = cnt_ref[p]

            def b16(j, c2):
                sl = j * 16 + lanes
                ee = plsc.load_gather(pbuf, [p * cap + sl])
                valid = sl < n
                dvv = jnp.where(valid, ee - (ee // PW) * PW, 0)
                ddv = jnp.where(valid, ee // PW, dump + sl)
                dbuf[0, pl.ds(j * 16, 16)] = ddv
                for l in range(16):
                    dvl = jnp.full((16,), dvv[l], jnp.int32)
                    for g in range(ng):
                        col = plsc.load_gather(pan, [g * 16 + lanes, dvl])
                        row[j * 16 + l, pl.ds(g * 16, 16)] = col
                return c2
            lax.fori_loop(0, (n + 15) // 16, b16, 0)
            cp = pltpu.async_copy(row, g_hbm.at[dbuf.at[0]], ssemA)
            if not defer:
                cp.wait()
            return carry
        return inner

    def wait_panel(tab, p, buf, psem):
        @pl.when(p < npan_full)
        def _():
            for r in range(e_dim // 8):
                pltpu.make_async_copy(
                    tab.at[pl.ds(r * 8, 8), pl.ds(vbase + p * PW, PW)],
                    buf.at[pl.ds(r * 8, 8), :], psem).wait()

    def prefetch(tab, p, buf, psem):
        @pl.when(p < npan_full)
        def _():
            for r in range(e_dim // 8):
                pltpu.async_copy(
                    tab.at[pl.ds(r * 8, 8), pl.ds(vbase + p * PW, PW)],
                    buf.at[pl.ds(r * 8, 8), :], psem)

    def sweep_table(tab, g_hbm, pbuf, cap, row, dbuf, dump, cnt_ref):
        srv = [serve(g_hbm, pbuf, cap, row, dbuf, dump, cnt_ref, pan, True)
               for pan in (panel, panel2)]
        prefetch(tab, 0, panel, psemA)

        def two(p2, carry):
            p = p2 * 2
            prefetch(tab, p + 1, panel2, psemB)
            wait_panel(tab, p, panel, psemA)

            @pl.when(p < npan_full)
            def _():
                srv[0](p, 0)
            prefetch(tab, p + 2, panel, psemA)
            wait_panel(tab, p + 1, panel2, psemB)

            @pl.when(p + 1 < npan_full)
            def _():
                srv[1](p + 1, 0)
            return carry
        lax.fori_loop(0, 32, two, 0)
        @pl.when(npan_full >= 1)
        def _():
            pltpu.make_async_copy(row, g_hbm.at[dbuf.at[0]], ssemA).wait()

    sweep_table(ttab, te_g, pt, PTCAP, trow, tdst, dump_t, pcnt_t)
    sweep_table(ctab, ce_g, pc, PCCAP, crow, cdst, dump_c, pcnt_c)

    @pl.when(o == 30)
    def _tail():
        pltpu.sync_copy(ttail, tailp)
        serve(te_g, pt, PTCAP, trow, tdst, dump_t, pcnt_t, tailp,
              False)(33, 0)
        pltpu.sync_copy(ctail, tailp)
        serve(ce_g, pc, PCCAP, crow, cdst, dump_c, pcnt_c, tailp,
              False)(33, 0)


def _dots_body(te_hbm, ce_hbm, out_hbm, te_v, ce_v, out_v, sem,
               *, b_per_w, b_dim, c_dim, e_dim):
    wid = lax.axis_index("s") * NC + lax.axis_index("c")
    lanes = lax.iota(jnp.int32, 16)
    cb = 128
    for chunk in range(b_per_w // cb):
        base = wid * b_per_w + chunk * cb
        copies = [pltpu.async_copy(te_hbm.at[pl.ds(base, cb)], te_v, sem)]
        for c in range(c_dim):
            copies.append(pltpu.async_copy(
                ce_hbm.at[pl.ds(c * b_dim + base, cb)],
                ce_v.at[pl.ds(c * cb, cb)], sem))
        for cp in copies:
            cp.wait()

        for blk in range(cb // 16):
            rows16 = blk * 16 + lanes
            zero = jnp.zeros((16,), jnp.float32)

            def ebody(e4, accs):
                for u in range(4):
                    e = e4 * 4 + u
                    ecol = jnp.full((16,), e, jnp.int32)
                    tg = plsc.load_gather(te_v, [rows16, ecol])
                    accs = tuple(
                        accs[c] + tg * plsc.load_gather(
                            ce_v, [c * cb + rows16, ecol])
                        for c in range(c_dim))
                return accs

            accs = lax.fori_loop(0, e_dim // 4, ebody, (zero,) * c_dim)
            for c in range(c_dim):
                out_v[c, pl.ds(blk * 16, 16)] = accs[c]

        pltpu.sync_copy(out_v, out_hbm.at[:, pl.ds(base, cb)])


def kernel(target, context, target_table, context_table):
    b_dim = target.shape[0]
    c_dim = context.shape[1]
    e_dim = target_table.shape[1]
    b_per_w = b_dim // NW

    ctx_t = context.T
    tt_t = target_table.T
    ct_t = context_table.T

    mesh = plsc.VectorSubcoreMesh(core_axis_name="c", subcore_axis_name="s")
    params = pltpu.CompilerParams(needs_layout_passes=False)

    bucketize = functools.partial(
        pl.kernel, mesh=mesh, compiler_params=params,
        out_type=(
            jax.ShapeDtypeStruct((NW, 32 * 2 * TCAP), jnp.int32),
            jax.ShapeDtypeStruct((NW, 32 * 2 * CCAP), jnp.int32),
            jax.ShapeDtypeStruct((NW * 8,), jnp.int32),
        ),
        scratch_types=[
            pltpu.VMEM((b_per_w,), jnp.int32),
            pltpu.VMEM((c_dim, b_per_w), jnp.int32),
            pltpu.VMEM((32 * 2 * TCAP,), jnp.int32),
            pltpu.VMEM((32 * 2 * CCAP,), jnp.int32),
            pltpu.SMEM((32,), jnp.int32),
            pltpu.SMEM((32,), jnp.int32),
            pltpu.SemaphoreType.DMA,
        ],
    )(functools.partial(_bucketize_body, b_per_w=b_per_w, c_dim=c_dim))
    tb, cb_, _sent = bucketize(target, ctx_t)

    sweep = functools.partial(
        pl.kernel, mesh=mesh, compiler_params=params,
        out_type=(
            jax.ShapeDtypeStruct((b_dim + PTCAP, 128), jnp.float32),
            jax.ShapeDtypeStruct((b_dim * c_dim + PCCAP, 128), jnp.float32),
        ),
        scratch_types=[
            pltpu.VMEM((e_dim, PW), jnp.float32),
            pltpu.VMEM((e_dim, PW), jnp.float32),
            pltpu.VMEM((e_dim, 64), jnp.float32),
            pltpu.VMEM((32, 2 * TCAP), jnp.int32),
            pltpu.VMEM((32, 2 * CCAP), jnp.int32),
            pltpu.VMEM((65 * PTCAP,), jnp.int32),
            pltpu.VMEM((65 * PCCAP,), jnp.int32),
            pltpu.VMEM((PTCAP, 128), jnp.float32),
            pltpu.VMEM((PCCAP, 128), jnp.float32),
            pltpu.VMEM((1, PTCAP), jnp.int32),
            pltpu.VMEM((1, PCCAP), jnp.int32),
            pltpu.SMEM((65,), jnp.int32),
            pltpu.SMEM((65,), jnp.int32),
            pltpu.SemaphoreType.DMA,
            pltpu.SemaphoreType.DMA,
            pltpu.SemaphoreType.DMA,
            pltpu.SemaphoreType.DMA,
        ],
    )(functools.partial(_sweep_body, e_dim=e_dim, b_dim=b_dim, c_dim=c_dim))
    vfull = (SPAN * 30) + ((target_table.shape[0] - SPAN * 30) // PW) * PW
    tt_tail = target_table[vfull:, :].T
    ct_tail = context_table[vfull:, :].T
    te_g, ce_g = sweep(tt_t, ct_t, tt_tail, ct_tail, tb, cb_)

    dots = functools.partial(
        pl.kernel, mesh=mesh, compiler_params=params,
        out_type=jax.ShapeDtypeStruct((c_dim, b_dim), jnp.float32),
        scratch_types=[
            pltpu.VMEM((128, 128), jnp.float32),
            pltpu.VMEM((c_dim * 128, 128), jnp.float32),
            pltpu.VMEM((c_dim, 128), jnp.float32),
            pltpu.SemaphoreType.DMA,
        ],
    )(functools.partial(_dots_body, b_per_w=b_per_w, b_dim=b_dim,
                        c_dim=c_dim, e_dim=e_dim))
    out = dots(te_g, ce_g)
    return out.T
